# Initial kernel scaffold; baseline (speedup 1.0000x reference)
import jax
import jax.numpy as jnp
from jax.experimental import pallas as pl


def kernel(x, shared_w1, shared_w2, shared_w3, routed_w1, routed_w2, routed_w3, router_w, expert_bias):
    raise NotImplementedError("write your pallas kernel here")


# speedup vs baseline: 1.2313x; 1.2313x over previous
import functools

import jax
import jax.numpy as jnp
from jax import lax
from jax.experimental import pallas as pl
from jax.experimental.pallas import tpu as pltpu
from jax.experimental.pallas import tpu_sc as plsc

E = 16
K = 2
D = 1024
H = 4096
RH = 1024
B_, S_ = 2, 2048
T = B_ * S_

BM_A = 256
BM = 256
CAP = K * T + E * BM
NB = CAP // BM

NC, NS, L = 2, 16, 16
NW = NC * NS


def _shared_router_body(x_ref, w1_ref, w2_ref, w3_ref, rw_ref, bias_ref,
                        shared_ref, gates_ref, sel_ref):
    xb = x_ref[...]
    cdims = (((1,), (1,)), ((), ()))
    a = lax.dot_general(xb, w1_ref[...], cdims, preferred_element_type=jnp.float32)
    b = lax.dot_general(xb, w2_ref[...], cdims, preferred_element_type=jnp.float32)
    hsw = (a * jax.nn.sigmoid(a)) * b
    shared_ref[...] = lax.dot_general(hsw, w3_ref[...], cdims,
                                      preferred_element_type=jnp.float32)
    logits = lax.dot_general(xb, rw_ref[...], cdims,
                             preferred_element_type=jnp.float32)
    scores = jax.nn.sigmoid(logits)
    selsc = scores + bias_ref[0, :]
    iota = lax.broadcasted_iota(jnp.int32, selsc.shape, 1)
    m1 = jnp.max(selsc, axis=-1, keepdims=True)
    i1 = jnp.min(jnp.where(selsc == m1, iota, E), axis=-1, keepdims=True)
    sel2 = jnp.where(iota == i1, -jnp.inf, selsc)
    m2 = jnp.max(sel2, axis=-1, keepdims=True)
    i2 = jnp.min(jnp.where(sel2 == m2, iota, E), axis=-1, keepdims=True)
    selmask = (iota == i1) | (iota == i2)
    gates_ref[...] = jnp.where(selmask, scores, 0.0)
    sel_ref[...] = selmask.astype(jnp.float32)


def _shared_router(x2d, w1, w2, w3, rw, bias2):
    nblk = T // BM_A
    return pl.pallas_call(
        _shared_router_body,
        grid=(nblk,),
        in_specs=[
            pl.BlockSpec((BM_A, D), lambda i: (i, 0)),
            pl.BlockSpec((H, D), lambda i: (0, 0)),
            pl.BlockSpec((H, D), lambda i: (0, 0)),
            pl.BlockSpec((D, H), lambda i: (0, 0)),
            pl.BlockSpec((E, D), lambda i: (0, 0)),
            pl.BlockSpec((1, E), lambda i: (0, 0)),
        ],
        out_specs=[
            pl.BlockSpec((BM_A, D), lambda i: (i, 0)),
            pl.BlockSpec((BM_A, E), lambda i: (i, 0)),
            pl.BlockSpec((BM_A, E), lambda i: (i, 0)),
        ],
        out_shape=[
            jax.ShapeDtypeStruct((T, D), jnp.float32),
            jax.ShapeDtypeStruct((T, E), jnp.float32),
            jax.ShapeDtypeStruct((T, E), jnp.float32),
        ],
    )(x2d, w1, w2, w3, rw, bias2)


RW_G = CAP // NW
CG = 32


@functools.partial(
    pl.kernel,
    mesh=plsc.VectorSubcoreMesh(core_axis_name="c", subcore_axis_name="s"),
    out_type=jax.ShapeDtypeStruct((CAP, D), jnp.float32),
    scratch_types=[
        pltpu.VMEM((CG,), jnp.int32),
        pltpu.VMEM((CG,), jnp.int32),
        pltpu.VMEM((CG, D), jnp.float32),
        pltpu.VMEM((CG, D), jnp.float32),
        pltpu.SemaphoreType.DMA,
        pltpu.SemaphoreType.DMA,
    ],
)
def _sc_gather(x_hbm, tok_hbm, xs_hbm, idx0, idx1, rows0, rows1, sem0, sem1):
    wid = lax.axis_index("s") * NC + lax.axis_index("c")
    base = wid * RW_G
    nch = RW_G // CG
    idx = (idx0, idx1)
    rows = (rows0, rows1)
    sem = (sem0, sem1)
    pltpu.sync_copy(tok_hbm.at[pl.ds(base, CG)], idx0)
    cps = {0: pltpu.async_copy(x_hbm.at[idx0], rows0, sem0)}
    for c in range(nch):
        if c + 1 < nch:
            j = (c + 1) % 2
            pltpu.sync_copy(tok_hbm.at[pl.ds(base + (c + 1) * CG, CG)], idx[j])
            cps[(c + 1) % 2] = pltpu.async_copy(x_hbm.at[idx[j]], rows[j], sem[j])
        cps[c % 2].wait()
        pltpu.sync_copy(rows[c % 2], xs_hbm.at[pl.ds(base + c * CG, CG), :])


def _grouped_body(meta_ref, xs_ref, w1_ref, w2_ref, w3_ref, g_ref, buf_ref):
    i = pl.program_id(0)

    @pl.when(i < meta_ref[NB])
    def _():
        xb = xs_ref[...]
        cdims = (((1,), (1,)), ((), ()))
        a = lax.dot_general(xb, w1_ref[0], cdims, preferred_element_type=jnp.float32)
        b = lax.dot_general(xb, w2_ref[0], cdims, preferred_element_type=jnp.float32)
        hsw = (a * jax.nn.sigmoid(a)) * b
        y = lax.dot_general(hsw, w3_ref[0], cdims, preferred_element_type=jnp.float32)
        buf_ref[...] = y * g_ref[0]


def _grouped(meta, xs, rw1, rw2, rw3, gates3):
    grid_spec = pltpu.PrefetchScalarGridSpec(
        num_scalar_prefetch=1,
        grid=(NB,),
        in_specs=[
            pl.BlockSpec((BM, D), lambda i, m: (i, 0)),
            pl.BlockSpec((1, RH, D), lambda i, m: (m[i], 0, 0)),
            pl.BlockSpec((1, RH, D), lambda i, m: (m[i], 0, 0)),
            pl.BlockSpec((1, D, RH), lambda i, m: (m[i], 0, 0)),
            pl.BlockSpec((1, BM, 1), lambda i, m: (i, 0, 0)),
        ],
        out_specs=pl.BlockSpec((BM, D), lambda i, m: (i, 0)),
    )
    return pl.pallas_call(
        _grouped_body,
        grid_spec=grid_spec,
        out_shape=jax.ShapeDtypeStruct((CAP, D), jnp.float32),
    )(meta, xs, rw1, rw2, rw3, gates3)


RW_C = T // NW
CC = 32


@functools.partial(
    pl.kernel,
    mesh=plsc.VectorSubcoreMesh(core_axis_name="c", subcore_axis_name="s"),
    out_type=jax.ShapeDtypeStruct((T, D), jnp.float32),
    scratch_types=[
        pltpu.VMEM((CC,), jnp.int32),
        pltpu.VMEM((CC,), jnp.int32),
        pltpu.VMEM((CC, D), jnp.float32),
        pltpu.VMEM((CC, D), jnp.float32),
        pltpu.VMEM((CC, D), jnp.float32),
        pltpu.SemaphoreType.DMA,
        pltpu.SemaphoreType.DMA,
        pltpu.SemaphoreType.DMA,
    ],
)
def _sc_combine(sh_hbm, buf_hbm, p0_hbm, p1_hbm, out_hbm,
                i0, i1, r0, r1, shv, sem0, sem1, sem2):
    wid = lax.axis_index("s") * NC + lax.axis_index("c")
    base = wid * RW_C
    for c in range(RW_C // CC):
        b0 = base + c * CC
        pltpu.sync_copy(p0_hbm.at[pl.ds(b0, CC)], i0)
        pltpu.sync_copy(p1_hbm.at[pl.ds(b0, CC)], i1)
        cp0 = pltpu.async_copy(buf_hbm.at[i0], r0, sem0)
        cp1 = pltpu.async_copy(buf_hbm.at[i1], r1, sem1)
        cp2 = pltpu.async_copy(sh_hbm.at[pl.ds(b0, CC)], shv, sem2)
        cp0.wait()
        cp1.wait()
        cp2.wait()

        def row_body(r, carry):
            for cc in range(D // L):
                sl = pl.ds(cc * L, L)
                shv[r, sl] = shv[r, sl] + r0[r, sl] + r1[r, sl]
            return carry

        lax.fori_loop(0, CC, row_body, 0)
        pltpu.sync_copy(shv, out_hbm.at[pl.ds(b0, CC), :])


def kernel(x, shared_w1, shared_w2, shared_w3, routed_w1, routed_w2,
           routed_w3, router_w, expert_bias):
    x2d = x.reshape(T, D)
    shared, gates, sel = _shared_router(
        x2d, shared_w1, shared_w2, shared_w3, router_w,
        expert_bias.reshape(1, E))

    m = sel > 0.5
    mi = m.astype(jnp.int32)
    rank = jnp.cumsum(mi, axis=0) - mi
    counts = jnp.sum(mi, axis=0)
    padded = ((counts + BM - 1) // BM) * BM
    cum = jnp.cumsum(padded)
    off = cum - padded
    total = cum[-1]
    p = off[None, :] + rank
    ppos = jnp.where(m, p, CAP).reshape(-1)
    tok = jnp.broadcast_to(
        jnp.arange(T, dtype=jnp.int32)[:, None], (T, E)).reshape(-1)
    sorted_tok = jnp.zeros((CAP + 1,), jnp.int32).at[ppos].set(tok)[:CAP]
    sorted_gate = jnp.zeros((CAP + 1,), jnp.float32).at[ppos].set(
        gates.reshape(-1))[:CAP]
    n_used = (total // BM).astype(jnp.int32)
    be_raw = jnp.clip(
        jnp.searchsorted(cum, jnp.arange(NB, dtype=jnp.int32) * BM,
                         side="right"), 0, E - 1).astype(jnp.int32)
    last_e = jnp.take(be_raw, jnp.maximum(n_used - 1, 0))
    be = jnp.where(jnp.arange(NB) < n_used, be_raw, last_e)
    meta = jnp.concatenate([be, n_used[None]])
    pos0 = jnp.min(jnp.where(m, p, CAP), axis=1).astype(jnp.int32)
    pos1 = jnp.max(jnp.where(m, p, -1), axis=1).astype(jnp.int32)

    xs = _sc_gather(x2d, sorted_tok)
    buf = _grouped(meta, xs, routed_w1, routed_w2, routed_w3,
                   sorted_gate.reshape(NB, BM, 1))
    out2d = _sc_combine(shared, buf, pos0, pos1)
    return out2d.reshape(B_, S_, D)
